# baseline (speedup 1.0000x reference)
import functools

import jax
import jax.numpy as jnp
from jax import lax
from jax.experimental import pallas as pl
from jax.experimental.pallas import tpu as pltpu
from jax.experimental.pallas import tpu_sc as plsc

N = 100000
E = 6400000
NC = 2
NS = 16
NW = NC * NS
CK = 128
NBUF = 3
CHUNKS_PER_W = 1563
NCHUNK = CHUNKS_PER_W * NW
EPAD = NCHUNK * CK
EROWS = EPAD // 128
LOOPN = CHUNKS_PER_W // NBUF
NA = N + 8
CNTF = 2 * N + 16
CNTW = 2 * N
RPT = 6256
CPT = 12512
HIGH = lax.Precision.HIGHEST

_mesh = functools.partial(
    plsc.VectorSubcoreMesh, core_axis_name="c", subcore_axis_name="s")
_SC_PARAMS = pltpu.CompilerParams(use_tc_tiling_on_sc=False)


def _per_tile(sid, total, per, op):
    @pl.when(sid < NS - 1)
    def _():
        op(sid * per, per)

    @pl.when(sid == NS - 1)
    def _():
        op((NS - 1) * per, total - (NS - 1) * per)


def _sc_scratch(rows_per_edge, with_cnt):
    sc = []
    for _ in range(NBUF):
        sc.append(pltpu.VMEM((CK,), jnp.int32))
        sc.append(pltpu.VMEM((CK,), jnp.int32))
        sc.append(pltpu.VMEM((CK, rows_per_edge), jnp.float32))
        if with_cnt:
            sc.append(pltpu.VMEM((CK,), jnp.int32))
            sc.append(pltpu.VMEM((CK,), jnp.float32))
    if with_cnt:
        sc.append(pltpu.VMEM((CPT,), jnp.float32))
        sc.append(pltpu.VMEM_SHARED((NA, rows_per_edge), jnp.float32))
        sc.append(pltpu.VMEM_SHARED((CNTF,), jnp.float32))
    else:
        sc.append(pltpu.VMEM_SHARED((NA, rows_per_edge), jnp.float32))
    for _ in range(3 * NBUF):
        sc.append(pltpu.SemaphoreType.DMA)
    return sc


@functools.partial(
    pl.kernel,
    mesh=_mesh(),
    out_type=(
        jax.ShapeDtypeStruct((NC, N, 8), jnp.float32),
        jax.ShapeDtypeStruct((NC * CNTW,), jnp.float32),
    ),
    scratch_types=_sc_scratch(8, True),
    compiler_params=_SC_PARAMS,
)
def _agg_pass_a(src_h, dst_h, tbl_h, z8_h, zc_h, acc_out, cnt_out, *sc):
    (s0, d0, r0, i0, v0,
     s1, d1, r1, i1, v1,
     s2, d2, r2, i2, v2,
     bounce, acc_s, cnt_s,
     ix0, ix1, ix2, g0, g1, g2, sc0, sc1, sc2) = sc
    srcv, dstv, rows = (s0, s1, s2), (d0, d1, d2), (r0, r1, r2)
    idxc, valv = (i0, i1, i2), (v0, v1, v2)
    ix_sem, g_sem, s_sem = (ix0, ix1, ix2), (g0, g1, g2), (sc0, sc1, sc2)
    cid = lax.axis_index("c")
    sid = lax.axis_index("s")
    wid = sid * NC + cid

    def _init(start, size):
        sl = pl.ds(start, size)
        pltpu.sync_copy(z8_h.at[sl], acc_s.at[sl])

    def _initc(start, size):
        pltpu.sync_copy(zc_h.at[pl.ds(start, size)], bounce.at[pl.ds(0, size)])
        pltpu.sync_copy(bounce.at[pl.ds(0, size)], cnt_s.at[pl.ds(start, size)])

    _per_tile(sid, NA, RPT, _init)
    _per_tile(sid, CNTF, CPT, _initc)
    plsc.subcore_barrier()

    def _row(t, u):
        return ((t * NBUF + u) * NW + wid) * CK

    for u in range(NBUF):
        pltpu.async_copy(src_h.at[pl.ds(_row(0, u), CK)], srcv[u], ix_sem[u])
        pltpu.async_copy(dst_h.at[pl.ds(_row(0, u), CK)], dstv[u], ix_sem[u])

    def body(t, carry):
        gds = []
        for u in range(NBUF):
            row = _row(t, u)
            pltpu.make_async_copy(
                src_h.at[pl.ds(row, CK)], srcv[u], ix_sem[u]).wait()
            pltpu.make_async_copy(
                dst_h.at[pl.ds(row, CK)], dstv[u], ix_sem[u]).wait()
            gds.append(pltpu.async_copy(tbl_h.at[srcv[u]], rows[u], g_sem[u]))
        one = jnp.ones((16,), jnp.float32)
        zero = jnp.zeros((16,), jnp.float32)
        for u in range(NBUF):
            for k in range(0, CK, 16):
                ksl = pl.ds(k, 16)
                s16 = srcv[u][ksl]
                d16 = dstv[u][ksl]
                hit = d16 < 2
                idxc[u][ksl] = jnp.where(hit, s16 * 2 + d16, 2 * N)
                valv[u][ksl] = jnp.where(hit, one, zero)
        sds = []
        for u in range(NBUF):
            gds[u].wait()
            sds.append(pltpu.async_copy(
                rows[u], acc_s.at[dstv[u]], s_sem[u], add=True))
            sds.append(pltpu.async_copy(
                valv[u], cnt_s.at[idxc[u]], s_sem[u], add=True))
        for d in sds:
            d.wait()

        @pl.when(t + 1 < LOOPN)
        def _():
            for u in range(NBUF):
                row = _row(t + 1, u)
                pltpu.async_copy(
                    src_h.at[pl.ds(row, CK)], srcv[u], ix_sem[u])
                pltpu.async_copy(
                    dst_h.at[pl.ds(row, CK)], dstv[u], ix_sem[u])
        return carry

    lax.fori_loop(0, LOOPN, body, 0)
    plsc.subcore_barrier()

    def _wout(start, size):
        sl = pl.ds(start, size)
        pltpu.sync_copy(acc_s.at[sl], acc_out.at[cid, sl])

    def _woutc(start, size):
        pltpu.sync_copy(cnt_s.at[pl.ds(start, size)], bounce.at[pl.ds(0, size)])
        pltpu.sync_copy(bounce.at[pl.ds(0, size)],
                        cnt_out.at[pl.ds(cid * CNTW + start, size)])

    _per_tile(sid, N, RPT, _wout)
    _per_tile(sid, CNTW, CPT, _woutc)


NH = N // 2
NHB = NH + 8
CPW_B = NCHUNK // NS
LOOPNB = CPW_B // NBUF
RPTB = 3128


@functools.partial(
    pl.kernel,
    mesh=_mesh(),
    out_type=jax.ShapeDtypeStruct((NC, NH, 16), jnp.float32),
    scratch_types=(
        [pltpu.VMEM((CK,), jnp.int32), pltpu.VMEM((CK,), jnp.int32),
         pltpu.VMEM((CK,), jnp.int32),
         pltpu.VMEM((CK, 16), jnp.float32)] * NBUF
        + [pltpu.VMEM_SHARED((NHB, 16), jnp.float32)]
        + [pltpu.SemaphoreType.DMA] * (3 * NBUF)
    ),
    compiler_params=_SC_PARAMS,
)
def _agg_pass_b(src_h, dst_h, tbl_h, zb_h, acc_out, *sc):
    (s0, d0, l0, r0, s1, d1, l1, r1, s2, d2, l2, r2, acc_s,
     ix0, ix1, ix2, g0, g1, g2, sc0, sc1, sc2) = sc
    srcv, dstv, rows = (s0, s1, s2), (d0, d1, d2), (r0, r1, r2)
    dloc = (l0, l1, l2)
    ix_sem, g_sem, s_sem = (ix0, ix1, ix2), (g0, g1, g2), (sc0, sc1, sc2)
    cid = lax.axis_index("c")
    sid = lax.axis_index("s")
    lo = cid * NH

    _per_tile(sid, NHB, RPTB, lambda start, size: pltpu.sync_copy(
        zb_h.at[pl.ds(start, size)], acc_s.at[pl.ds(start, size)]))
    plsc.subcore_barrier()

    def _row(t, u):
        return ((t * NBUF + u) * NS + sid) * CK

    for u in range(NBUF):
        pltpu.async_copy(src_h.at[pl.ds(_row(0, u), CK)], srcv[u], ix_sem[u])
        pltpu.async_copy(dst_h.at[pl.ds(_row(0, u), CK)], dstv[u], ix_sem[u])

    def body(t, carry):
        gds = []
        for u in range(NBUF):
            row = _row(t, u)
            pltpu.make_async_copy(
                src_h.at[pl.ds(row, CK)], srcv[u], ix_sem[u]).wait()
            pltpu.make_async_copy(
                dst_h.at[pl.ds(row, CK)], dstv[u], ix_sem[u]).wait()
            gds.append(pltpu.async_copy(tbl_h.at[srcv[u]], rows[u], g_sem[u]))
        for u in range(NBUF):
            for k in range(0, CK, 16):
                ksl = pl.ds(k, 16)
                d_loc = dstv[u][ksl] - lo
                owned = (d_loc >= 0) & (d_loc < NH)
                dloc[u][ksl] = jnp.where(owned, d_loc, NH)
        sds = []
        for u in range(NBUF):
            gds[u].wait()
            sds.append(pltpu.async_copy(
                rows[u], acc_s.at[dloc[u]], s_sem[u], add=True))
        for d in sds:
            d.wait()

        @pl.when(t + 1 < LOOPNB)
        def _():
            for u in range(NBUF):
                row = _row(t + 1, u)
                pltpu.async_copy(
                    src_h.at[pl.ds(row, CK)], srcv[u], ix_sem[u])
                pltpu.async_copy(
                    dst_h.at[pl.ds(row, CK)], dstv[u], ix_sem[u])
        return carry

    lax.fori_loop(0, LOOPNB, body, 0)
    plsc.subcore_barrier()
    _per_tile(sid, NH, RPTB, lambda start, size: pltpu.sync_copy(
        acc_s.at[pl.ds(start, size)], acc_out.at[cid, pl.ds(start, size)]))


BD = 2000
G = N // BD


def _dense1_body(acc_ref, cntp_ref, x_ref, wl_ref, wr_ref, b_ref,
                 h1p_ref, cnt_ref):
    acc = acc_ref[...]
    agg = acc[0, :, :6] + acc[1, :, :6]
    deg = acc[0, :, 6] + acc[1, :, 6]
    dinv = 1.0 / jnp.clip(deg, 1.0, None)
    mean = agg * dinv[:, None]
    xb = x_ref[...][:, :6]
    h1 = lax.dot_general(mean, wl_ref[...], (((1,), (1,)), ((), ())),
                         precision=HIGH, preferred_element_type=jnp.float32)
    h1 = h1 + lax.dot_general(xb, wr_ref[...], (((1,), (1,)), ((), ())),
                              precision=HIGH,
                              preferred_element_type=jnp.float32)
    h1 = jnp.maximum(h1 + b_ref[...], 0.0)
    h1p_ref[...] = jnp.concatenate(
        [h1, dinv[:, None], jnp.zeros((BD, 3), jnp.float32)], axis=1)
    cntp = cntp_ref[...]
    cnt_ref[...] = cntp[0] + cntp[1]


def _dense2_body(acc_ref, h1p_ref, cnt_ref, w2l_ref, w2r_ref, b2_ref,
                 w3l_ref, w3r_ref, b3_ref, out_ref, agg3_s, h2f_s, dinv_s):
    i = pl.program_id(0)
    acc = acc_ref[...]
    h1p = h1p_ref[...]
    dinv = h1p[:, 12]
    mean2 = acc[:, :12] * dinv[:, None]
    h2 = lax.dot_general(mean2, w2l_ref[...], (((1,), (1,)), ((), ())),
                         precision=HIGH, preferred_element_type=jnp.float32)
    h2 = h2 + lax.dot_general(h1p[:, :12], w2r_ref[...],
                              (((1,), (1,)), ((), ())),
                              precision=HIGH,
                              preferred_element_type=jnp.float32)
    h2 = jnp.maximum(h2 + b2_ref[...], 0.0)
    part = lax.dot_general(cnt_ref[...], h2, (((0,), (0,)), ((), ())),
                           precision=HIGH, preferred_element_type=jnp.float32)

    @pl.when(i == 0)
    def _():
        agg3_s[...] = part
        h2f_s[...] = h2[0:2, :]
        dinv_s[...] = dinv[0:2][:, None]

    @pl.when(i > 0)
    def _():
        agg3_s[...] = agg3_s[...] + part

    @pl.when(i == G - 1)
    def _():
        mean3 = agg3_s[...] * dinv_s[...]
        o = lax.dot_general(mean3, w3l_ref[...], (((1,), (1,)), ((), ())),
                            precision=HIGH, preferred_element_type=jnp.float32)
        o = o + lax.dot_general(h2f_s[...], w3r_ref[...],
                                (((1,), (1,)), ((), ())),
                                precision=HIGH,
                                preferred_element_type=jnp.float32)
        out_ref[...] = o + b3_ref[...]


_dense1 = pl.pallas_call(
    _dense1_body,
    grid=(G,),
    in_specs=[
        pl.BlockSpec((2, BD, 8), lambda i: (0, i, 0)),
        pl.BlockSpec((2, BD, 2), lambda i: (0, i, 0)),
        pl.BlockSpec((BD, 8), lambda i: (i, 0)),
        pl.BlockSpec((12, 6), lambda i: (0, 0)),
        pl.BlockSpec((12, 6), lambda i: (0, 0)),
        pl.BlockSpec((1, 12), lambda i: (0, 0)),
    ],
    out_specs=[
        pl.BlockSpec((BD, 16), lambda i: (i, 0)),
        pl.BlockSpec((BD, 2), lambda i: (i, 0)),
    ],
    out_shape=[
        jax.ShapeDtypeStruct((N, 16), jnp.float32),
        jax.ShapeDtypeStruct((N, 2), jnp.float32),
    ],
)

_dense2 = pl.pallas_call(
    _dense2_body,
    grid=(G,),
    in_specs=[
        pl.BlockSpec((BD, 16), lambda i: (i, 0)),
        pl.BlockSpec((BD, 16), lambda i: (i, 0)),
        pl.BlockSpec((BD, 2), lambda i: (i, 0)),
        pl.BlockSpec((24, 12), lambda i: (0, 0)),
        pl.BlockSpec((24, 12), lambda i: (0, 0)),
        pl.BlockSpec((1, 24), lambda i: (0, 0)),
        pl.BlockSpec((6, 24), lambda i: (0, 0)),
        pl.BlockSpec((6, 24), lambda i: (0, 0)),
        pl.BlockSpec((1, 6), lambda i: (0, 0)),
    ],
    out_specs=pl.BlockSpec((2, 6), lambda i: (0, 0)),
    out_shape=jax.ShapeDtypeStruct((2, 6), jnp.float32),
    scratch_shapes=[
        pltpu.VMEM((2, 24), jnp.float32),
        pltpu.VMEM((2, 24), jnp.float32),
        pltpu.VMEM((2, 1), jnp.float32),
    ],
)


def kernel(x, edge_index, W1l, W1r, b1, W2l, W2r, b2, W3l, W3r, b3):
    npad = EPAD - E
    src = jnp.concatenate([edge_index[0], jnp.zeros((npad,), jnp.int32)])
    dst = jnp.concatenate([edge_index[1], jnp.full((npad,), N, jnp.int32)])
    xpad = jnp.concatenate(
        [x, jnp.ones((N, 1), jnp.float32), jnp.zeros((N, 1), jnp.float32)],
        axis=1)
    z8 = jnp.zeros((NA, 8), jnp.float32)
    zc = jnp.zeros((CNTF,), jnp.float32)
    zb = jnp.zeros((NHB, 16), jnp.float32)

    accA, cntA = _agg_pass_a(src, dst, xpad, z8, zc)
    cntp = cntA.reshape(NC, N, 2)
    h1pad, cnt = _dense1(accA, cntp, xpad, W1l, W1r, b1.reshape(1, 12))
    accB = _agg_pass_b(src, dst, h1pad, zb)
    out01 = _dense2(accB.reshape(N, 16), h1pad, cnt, W2l, W2r,
                    b2.reshape(1, 24), W3l, W3r, b3.reshape(1, 6))
    return (out01[0], out01[1])


# speedup vs baseline: 2.5009x; 2.5009x over previous
import functools

import jax
import jax.numpy as jnp
from jax import lax
from jax.experimental import pallas as pl
from jax.experimental.pallas import tpu as pltpu
from jax.experimental.pallas import tpu_sc as plsc

N = 100000
E = 6400000
NC = 2
NS = 16
NW = NC * NS
CK = 128
NBUF = 3
CHUNKS_PER_W = 1563
NCHUNK = CHUNKS_PER_W * NW
EPAD = NCHUNK * CK
EROWS = EPAD // 128
LOOPN = CHUNKS_PER_W // NBUF
NA = N + 8
CNTF = 2 * N + 16
CNTW = 2 * N
RPT = 6256
CPT = 12512
HIGH = lax.Precision.HIGHEST

_mesh = functools.partial(
    plsc.VectorSubcoreMesh, core_axis_name="c", subcore_axis_name="s")
_SC_PARAMS = pltpu.CompilerParams(use_tc_tiling_on_sc=False)


def _per_tile(sid, total, per, op):
    @pl.when(sid < NS - 1)
    def _():
        op(sid * per, per)

    @pl.when(sid == NS - 1)
    def _():
        op((NS - 1) * per, total - (NS - 1) * per)


def _sc_scratch(rows_per_edge, with_cnt):
    sc = []
    for _ in range(NBUF):
        sc.append(pltpu.VMEM((CK,), jnp.int32))
        sc.append(pltpu.VMEM((CK,), jnp.int32))
        sc.append(pltpu.VMEM((CK, rows_per_edge), jnp.float32))
        if with_cnt:
            sc.append(pltpu.VMEM((CK,), jnp.int32))
            sc.append(pltpu.VMEM((CK,), jnp.float32))
    if with_cnt:
        sc.append(pltpu.VMEM((CPT,), jnp.float32))
        sc.append(pltpu.VMEM_SHARED((NA, rows_per_edge), jnp.float32))
        sc.append(pltpu.VMEM_SHARED((CNTF,), jnp.float32))
    else:
        sc.append(pltpu.VMEM_SHARED((NA, rows_per_edge), jnp.float32))
    for _ in range(3 * NBUF):
        sc.append(pltpu.SemaphoreType.DMA)
    return sc


@functools.partial(
    pl.kernel,
    mesh=_mesh(),
    out_type=(
        jax.ShapeDtypeStruct((NC, N, 8), jnp.float32),
        jax.ShapeDtypeStruct((NC * CNTW,), jnp.float32),
    ),
    scratch_types=_sc_scratch(8, True),
    compiler_params=_SC_PARAMS,
)
def _agg_pass_a(src_h, dst_h, tbl_h, z8_h, zc_h, acc_out, cnt_out, *sc):
    (s0, d0, r0, i0, v0,
     s1, d1, r1, i1, v1,
     s2, d2, r2, i2, v2,
     bounce, acc_s, cnt_s,
     ix0, ix1, ix2, g0, g1, g2, sc0, sc1, sc2) = sc
    srcv, dstv, rows = (s0, s1, s2), (d0, d1, d2), (r0, r1, r2)
    idxc, valv = (i0, i1, i2), (v0, v1, v2)
    ix_sem, g_sem, s_sem = (ix0, ix1, ix2), (g0, g1, g2), (sc0, sc1, sc2)
    cid = lax.axis_index("c")
    sid = lax.axis_index("s")
    wid = sid * NC + cid

    def _init(start, size):
        sl = pl.ds(start, size)
        pltpu.sync_copy(z8_h.at[sl], acc_s.at[sl])

    def _initc(start, size):
        pltpu.sync_copy(zc_h.at[pl.ds(start, size)], bounce.at[pl.ds(0, size)])
        pltpu.sync_copy(bounce.at[pl.ds(0, size)], cnt_s.at[pl.ds(start, size)])

    _per_tile(sid, NA, RPT, _init)
    _per_tile(sid, CNTF, CPT, _initc)
    plsc.subcore_barrier()

    def _row(t, u):
        return ((t * NBUF + u) * NW + wid) * CK

    for u in range(NBUF):
        pltpu.async_copy(src_h.at[pl.ds(_row(0, u), CK)], srcv[u], ix_sem[u])
        pltpu.async_copy(dst_h.at[pl.ds(_row(0, u), CK)], dstv[u], ix_sem[u])

    def body(t, carry):
        gds = []
        for u in range(NBUF):
            row = _row(t, u)
            pltpu.make_async_copy(
                src_h.at[pl.ds(row, CK)], srcv[u], ix_sem[u]).wait()
            pltpu.make_async_copy(
                dst_h.at[pl.ds(row, CK)], dstv[u], ix_sem[u]).wait()
            gds.append(pltpu.async_copy(tbl_h.at[srcv[u]], rows[u], g_sem[u]))
        one = jnp.ones((16,), jnp.float32)
        zero = jnp.zeros((16,), jnp.float32)
        for u in range(NBUF):
            for k in range(0, CK, 16):
                ksl = pl.ds(k, 16)
                s16 = srcv[u][ksl]
                d16 = dstv[u][ksl]
                hit = d16 < 2
                idxc[u][ksl] = jnp.where(hit, s16 * 2 + d16, -1)
                valv[u][ksl] = jnp.where(hit, one, zero)
        sds = []
        for u in range(NBUF):
            gds[u].wait()
            sds.append(pltpu.async_copy(
                rows[u], acc_s.at[plsc.Indices(dstv[u], ignored_value=N)],
                s_sem[u], add=True))
            sds.append(pltpu.async_copy(
                valv[u], cnt_s.at[plsc.Indices(idxc[u], ignored_value=-1)],
                s_sem[u], add=True))
        for d in sds:
            d.wait()

        @pl.when(t + 1 < LOOPN)
        def _():
            for u in range(NBUF):
                row = _row(t + 1, u)
                pltpu.async_copy(
                    src_h.at[pl.ds(row, CK)], srcv[u], ix_sem[u])
                pltpu.async_copy(
                    dst_h.at[pl.ds(row, CK)], dstv[u], ix_sem[u])
        return carry

    lax.fori_loop(0, LOOPN, body, 0)
    plsc.subcore_barrier()

    def _wout(start, size):
        sl = pl.ds(start, size)
        pltpu.sync_copy(acc_s.at[sl], acc_out.at[cid, sl])

    def _woutc(start, size):
        pltpu.sync_copy(cnt_s.at[pl.ds(start, size)], bounce.at[pl.ds(0, size)])
        pltpu.sync_copy(bounce.at[pl.ds(0, size)],
                        cnt_out.at[pl.ds(cid * CNTW + start, size)])

    _per_tile(sid, N, RPT, _wout)
    _per_tile(sid, CNTW, CPT, _woutc)


NH = N // 2
NHB = NH + 8
CPW_B = NCHUNK // NS
LOOPNB = CPW_B // NBUF
RPTB = 3128


@functools.partial(
    pl.kernel,
    mesh=_mesh(),
    out_type=jax.ShapeDtypeStruct((NC, NH, 16), jnp.float32),
    scratch_types=(
        [pltpu.VMEM((CK,), jnp.int32), pltpu.VMEM((CK,), jnp.int32),
         pltpu.VMEM((CK,), jnp.int32), pltpu.VMEM((CK,), jnp.int32),
         pltpu.VMEM((CK, 16), jnp.float32)] * NBUF
        + [pltpu.VMEM_SHARED((NHB, 16), jnp.float32)]
        + [pltpu.SemaphoreType.DMA] * (3 * NBUF)
    ),
    compiler_params=_SC_PARAMS,
)
def _agg_pass_b(src_h, dst_h, tbl_h, zb_h, acc_out, *sc):
    (s0, d0, l0, m0, r0, s1, d1, l1, m1, r1, s2, d2, l2, m2, r2, acc_s,
     ix0, ix1, ix2, g0, g1, g2, sc0, sc1, sc2) = sc
    srcv, dstv, rows = (s0, s1, s2), (d0, d1, d2), (r0, r1, r2)
    dloc = (l0, l1, l2)
    sgat = (m0, m1, m2)
    ix_sem, g_sem, s_sem = (ix0, ix1, ix2), (g0, g1, g2), (sc0, sc1, sc2)
    cid = lax.axis_index("c")
    sid = lax.axis_index("s")
    lo = cid * NH

    _per_tile(sid, NHB, RPTB, lambda start, size: pltpu.sync_copy(
        zb_h.at[pl.ds(start, size)], acc_s.at[pl.ds(start, size)]))
    plsc.subcore_barrier()

    def _row(t, u):
        return ((t * NBUF + u) * NS + sid) * CK

    for u in range(NBUF):
        pltpu.async_copy(src_h.at[pl.ds(_row(0, u), CK)], srcv[u], ix_sem[u])
        pltpu.async_copy(dst_h.at[pl.ds(_row(0, u), CK)], dstv[u], ix_sem[u])

    def body(t, carry):
        gds = []
        for u in range(NBUF):
            row = _row(t, u)
            pltpu.make_async_copy(
                src_h.at[pl.ds(row, CK)], srcv[u], ix_sem[u]).wait()
            pltpu.make_async_copy(
                dst_h.at[pl.ds(row, CK)], dstv[u], ix_sem[u]).wait()
            for k in range(0, CK, 16):
                ksl = pl.ds(k, 16)
                d_loc = dstv[u][ksl] - lo
                owned = (d_loc >= 0) & (d_loc < NH)
                dloc[u][ksl] = jnp.where(owned, d_loc, -1)
                sgat[u][ksl] = jnp.where(owned, srcv[u][ksl], -1)
            gds.append(pltpu.async_copy(
                tbl_h.at[plsc.Indices(sgat[u], ignored_value=-1)],
                rows[u], g_sem[u]))
        sds = []
        for u in range(NBUF):
            gds[u].wait()
            sds.append(pltpu.async_copy(
                rows[u], acc_s.at[plsc.Indices(dloc[u], ignored_value=-1)],
                s_sem[u], add=True))
        for d in sds:
            d.wait()

        @pl.when(t + 1 < LOOPNB)
        def _():
            for u in range(NBUF):
                row = _row(t + 1, u)
                pltpu.async_copy(
                    src_h.at[pl.ds(row, CK)], srcv[u], ix_sem[u])
                pltpu.async_copy(
                    dst_h.at[pl.ds(row, CK)], dstv[u], ix_sem[u])
        return carry

    lax.fori_loop(0, LOOPNB, body, 0)
    plsc.subcore_barrier()
    _per_tile(sid, NH, RPTB, lambda start, size: pltpu.sync_copy(
        acc_s.at[pl.ds(start, size)], acc_out.at[cid, pl.ds(start, size)]))


BD = 2000
G = N // BD


def _dense1_body(acc_ref, cntp_ref, x_ref, wl_ref, wr_ref, b_ref,
                 h1p_ref, cnt_ref):
    acc = acc_ref[...]
    agg = acc[0, :, :6] + acc[1, :, :6]
    deg = acc[0, :, 6] + acc[1, :, 6]
    dinv = 1.0 / jnp.clip(deg, 1.0, None)
    mean = agg * dinv[:, None]
    xb = x_ref[...][:, :6]
    h1 = lax.dot_general(mean, wl_ref[...], (((1,), (1,)), ((), ())),
                         precision=HIGH, preferred_element_type=jnp.float32)
    h1 = h1 + lax.dot_general(xb, wr_ref[...], (((1,), (1,)), ((), ())),
                              precision=HIGH,
                              preferred_element_type=jnp.float32)
    h1 = jnp.maximum(h1 + b_ref[...], 0.0)
    h1p_ref[...] = jnp.concatenate(
        [h1, dinv[:, None], jnp.zeros((BD, 3), jnp.float32)], axis=1)
    cntp = cntp_ref[...]
    cnt_ref[...] = cntp[0] + cntp[1]


def _dense2_body(acc_ref, h1p_ref, cnt_ref, w2l_ref, w2r_ref, b2_ref,
                 w3l_ref, w3r_ref, b3_ref, out_ref, agg3_s, h2f_s, dinv_s):
    i = pl.program_id(0)
    acc = acc_ref[...]
    h1p = h1p_ref[...]
    dinv = h1p[:, 12]
    mean2 = acc[:, :12] * dinv[:, None]
    h2 = lax.dot_general(mean2, w2l_ref[...], (((1,), (1,)), ((), ())),
                         precision=HIGH, preferred_element_type=jnp.float32)
    h2 = h2 + lax.dot_general(h1p[:, :12], w2r_ref[...],
                              (((1,), (1,)), ((), ())),
                              precision=HIGH,
                              preferred_element_type=jnp.float32)
    h2 = jnp.maximum(h2 + b2_ref[...], 0.0)
    part = lax.dot_general(cnt_ref[...], h2, (((0,), (0,)), ((), ())),
                           precision=HIGH, preferred_element_type=jnp.float32)

    @pl.when(i == 0)
    def _():
        agg3_s[...] = part
        h2f_s[...] = h2[0:2, :]
        dinv_s[...] = dinv[0:2][:, None]

    @pl.when(i > 0)
    def _():
        agg3_s[...] = agg3_s[...] + part

    @pl.when(i == G - 1)
    def _():
        mean3 = agg3_s[...] * dinv_s[...]
        o = lax.dot_general(mean3, w3l_ref[...], (((1,), (1,)), ((), ())),
                            precision=HIGH, preferred_element_type=jnp.float32)
        o = o + lax.dot_general(h2f_s[...], w3r_ref[...],
                                (((1,), (1,)), ((), ())),
                                precision=HIGH,
                                preferred_element_type=jnp.float32)
        out_ref[...] = o + b3_ref[...]


_dense1 = pl.pallas_call(
    _dense1_body,
    grid=(G,),
    in_specs=[
        pl.BlockSpec((2, BD, 8), lambda i: (0, i, 0)),
        pl.BlockSpec((2, BD, 2), lambda i: (0, i, 0)),
        pl.BlockSpec((BD, 8), lambda i: (i, 0)),
        pl.BlockSpec((12, 6), lambda i: (0, 0)),
        pl.BlockSpec((12, 6), lambda i: (0, 0)),
        pl.BlockSpec((1, 12), lambda i: (0, 0)),
    ],
    out_specs=[
        pl.BlockSpec((BD, 16), lambda i: (i, 0)),
        pl.BlockSpec((BD, 2), lambda i: (i, 0)),
    ],
    out_shape=[
        jax.ShapeDtypeStruct((N, 16), jnp.float32),
        jax.ShapeDtypeStruct((N, 2), jnp.float32),
    ],
)

_dense2 = pl.pallas_call(
    _dense2_body,
    grid=(G,),
    in_specs=[
        pl.BlockSpec((BD, 16), lambda i: (i, 0)),
        pl.BlockSpec((BD, 16), lambda i: (i, 0)),
        pl.BlockSpec((BD, 2), lambda i: (i, 0)),
        pl.BlockSpec((24, 12), lambda i: (0, 0)),
        pl.BlockSpec((24, 12), lambda i: (0, 0)),
        pl.BlockSpec((1, 24), lambda i: (0, 0)),
        pl.BlockSpec((6, 24), lambda i: (0, 0)),
        pl.BlockSpec((6, 24), lambda i: (0, 0)),
        pl.BlockSpec((1, 6), lambda i: (0, 0)),
    ],
    out_specs=pl.BlockSpec((2, 6), lambda i: (0, 0)),
    out_shape=jax.ShapeDtypeStruct((2, 6), jnp.float32),
    scratch_shapes=[
        pltpu.VMEM((2, 24), jnp.float32),
        pltpu.VMEM((2, 24), jnp.float32),
        pltpu.VMEM((2, 1), jnp.float32),
    ],
)


def kernel(x, edge_index, W1l, W1r, b1, W2l, W2r, b2, W3l, W3r, b3):
    npad = EPAD - E
    src = jnp.concatenate([edge_index[0], jnp.zeros((npad,), jnp.int32)])
    dst = jnp.concatenate([edge_index[1], jnp.full((npad,), N, jnp.int32)])
    xpad = jnp.concatenate(
        [x, jnp.ones((N, 1), jnp.float32), jnp.zeros((N, 1), jnp.float32)],
        axis=1)
    z8 = jnp.zeros((NA, 8), jnp.float32)
    zc = jnp.zeros((CNTF,), jnp.float32)
    zb = jnp.zeros((NHB, 16), jnp.float32)

    accA, cntA = _agg_pass_a(src, dst, xpad, z8, zc)
    cntp = cntA.reshape(NC, N, 2)
    h1pad, cnt = _dense1(accA, cntp, xpad, W1l, W1r, b1.reshape(1, 12))
    accB = _agg_pass_b(src, dst, h1pad, zb)
    out01 = _dense2(accB.reshape(N, 16), h1pad, cnt, W2l, W2r,
                    b2.reshape(1, 24), W3l, W3r, b3.reshape(1, 6))
    return (out01[0], out01[1])
